# baseline (speedup 1.0000x reference)
import functools

import jax
import jax.numpy as jnp
from jax import lax
from jax.experimental import pallas as pl
from jax.experimental.pallas import tpu as pltpu
from jax.experimental.pallas import tpu_sc as plsc

DIM = 64
CHUNK = 128
NSLOT = 2


def _rope_gather(position_ids, fused_table):
    bsz, seq = position_ids.shape
    n_total = bsz * seq
    info = plsc.get_sparse_core_info()
    nw = info.num_cores * info.num_subcores
    n_per_w = n_total // nw
    n_ch = n_per_w // CHUNK
    w_per_b = nw // bsz

    mesh = plsc.VectorSubcoreMesh(core_axis_name="c", subcore_axis_name="s")

    @functools.partial(
        pl.kernel,
        mesh=mesh,
        compiler_params=pltpu.CompilerParams(
            use_tc_tiling_on_sc=True, needs_layout_passes=False),
        out_type=(
            jax.ShapeDtypeStruct((bsz, DIM, seq), jnp.float32),
            jax.ShapeDtypeStruct((bsz, DIM, seq), jnp.float32),
        ),
        scratch_types=[
            pltpu.VMEM((n_per_w,), jnp.int32),
            pltpu.VMEM((NSLOT, CHUNK, 2 * DIM), jnp.float32),
            pltpu.VMEM((NSLOT, DIM, CHUNK), jnp.float32),
            pltpu.VMEM((NSLOT, DIM, CHUNK), jnp.float32),
            pltpu.SemaphoreType.DMA((NSLOT,)),
            pltpu.SemaphoreType.DMA((NSLOT,)),
            pltpu.SemaphoreType.DMA((NSLOT,)),
        ],
    )
    def k(tab_hbm, idx_hbm, cos_out, sin_out, idx_v, buf, ct, st, gsem, csem, ssem):
        wid = lax.axis_index("s") * info.num_cores + lax.axis_index("c")
        b = wid // w_per_b
        s_base = (wid % w_per_b) * n_per_w
        pltpu.sync_copy(idx_hbm.at[b, pl.ds(s_base, n_per_w)], idx_v)

        iota = lax.iota(jnp.int32, 16)
        gops = [None] * NSLOT
        cops = [None] * NSLOT
        sops = [None] * NSLOT

        def refill(r):
            sl = r % NSLOT
            gops[sl] = pltpu.async_copy(tab_hbm.at[idx_v.at[pl.ds(r * CHUNK, CHUNK)]],
                                        buf.at[sl], gsem.at[sl])

        def transpose(sl):
            src = buf.at[sl]
            cdst = ct.at[sl]
            sdst = st.at[sl]

            def body(d, carry):
                cold = jnp.full((16,), d, jnp.int32)
                sold = cold + DIM
                for g in range(8):
                    rows = iota + (g * 16)
                    cdst[d, pl.ds(g * 16, 16)] = plsc.load_gather(src, [rows, cold])
                    sdst[d, pl.ds(g * 16, 16)] = plsc.load_gather(src, [rows, sold])
                return carry

            lax.fori_loop(0, DIM, body, 0, unroll=2)

        refill(0)
        for j in range(n_ch):
            sl = j % NSLOT
            gops[sl].wait()
            if j + 1 < n_ch:
                nsl = (j + 1) % NSLOT
                if cops[nsl] is not None:
                    cops[nsl].wait()
                    sops[nsl].wait()
                refill(j + 1)
            transpose(sl)
            s0 = s_base + j * CHUNK
            cops[sl] = pltpu.async_copy(ct.at[sl], cos_out.at[b, :, pl.ds(s0, CHUNK)],
                                        csem.at[sl])
            sops[sl] = pltpu.async_copy(st.at[sl], sin_out.at[b, :, pl.ds(s0, CHUNK)],
                                        ssem.at[sl])
        for sl in range(NSLOT):
            if cops[sl] is not None:
                cops[sl].wait()
                sops[sl].wait()

    return k(fused_table, position_ids)


def kernel(x, position_ids, cos_cached, sin_cached):
    fused = jnp.concatenate([cos_cached, sin_cached], axis=-1)
    cos_t, sin_t = _rope_gather(position_ids, fused)
    cos = jnp.swapaxes(cos_t, 1, 2).astype(x.dtype)
    sin = jnp.swapaxes(sin_t, 1, 2).astype(x.dtype)
    return cos, sin


# speedup vs baseline: 1.5774x; 1.5774x over previous
import functools

import jax
import jax.numpy as jnp
from jax import lax
from jax.experimental import pallas as pl
from jax.experimental.pallas import tpu as pltpu
from jax.experimental.pallas import tpu_sc as plsc

DIM = 64
HALF = 32
CHUNK = 128
NSLOT = 2


def _rope_gather(position_ids, fused_table):
    bsz, seq = position_ids.shape
    n_total = bsz * seq
    info = plsc.get_sparse_core_info()
    nw = info.num_cores * info.num_subcores
    n_per_w = n_total // nw
    n_ch = n_per_w // CHUNK
    w_per_b = nw // bsz

    mesh = plsc.VectorSubcoreMesh(core_axis_name="c", subcore_axis_name="s")

    @functools.partial(
        pl.kernel,
        mesh=mesh,
        compiler_params=pltpu.CompilerParams(
            use_tc_tiling_on_sc=True, needs_layout_passes=False),
        out_type=(
            jax.ShapeDtypeStruct((bsz, DIM, seq), jnp.float32),
            jax.ShapeDtypeStruct((bsz, DIM, seq), jnp.float32),
        ),
        scratch_types=[
            pltpu.VMEM((n_per_w,), jnp.int32),
            pltpu.VMEM((NSLOT, CHUNK, 2 * DIM), jnp.float32),
            pltpu.VMEM((NSLOT, HALF, CHUNK), jnp.float32),
            pltpu.VMEM((NSLOT, HALF, CHUNK), jnp.float32),
            pltpu.SemaphoreType.DMA((NSLOT,)),
            pltpu.SemaphoreType.DMA((NSLOT,)),
            pltpu.SemaphoreType.DMA((NSLOT,)),
        ],
    )
    def k(tab_hbm, idx_hbm, cos_out, sin_out, idx_v, buf, ct, st, gsem, csem, ssem):
        wid = lax.axis_index("s") * info.num_cores + lax.axis_index("c")
        b = wid // w_per_b
        s_base = (wid % w_per_b) * n_per_w
        pltpu.sync_copy(idx_hbm.at[b, pl.ds(s_base, n_per_w)], idx_v)

        iota = lax.iota(jnp.int32, 16)
        gops = [None] * NSLOT
        wops = [[] for _ in range(NSLOT)]

        def refill(r):
            sl = r % NSLOT
            gops[sl] = pltpu.async_copy(
                tab_hbm.at[idx_v.at[pl.ds(r * CHUNK, CHUNK)]], buf.at[sl],
                gsem.at[sl])

        def transpose(sl):
            src = buf.at[sl]
            cdst = ct.at[sl]
            sdst = st.at[sl]

            def body(t, carry):
                d0 = t * 4
                for dd in range(4):
                    d = d0 + dd
                    ccol = jnp.zeros((16,), jnp.int32) + d
                    scol = ccol + DIM
                    for g in range(8):
                        rows = iota + (g * 16)
                        cdst[d, pl.ds(g * 16, 16)] = plsc.load_gather(
                            src, [rows, ccol])
                        sdst[d, pl.ds(g * 16, 16)] = plsc.load_gather(
                            src, [rows, scol])
                return carry

            lax.fori_loop(0, HALF // 4, body, 0)

        refill(0)
        for j in range(n_ch):
            sl = j % NSLOT
            gops[sl].wait()
            if j + 1 < n_ch:
                refill(j + 1)
            for op in wops[sl]:
                op.wait()
            wops[sl] = []
            transpose(sl)
            s0 = s_base + j * CHUNK
            for half in range(2):
                wops[sl].append(pltpu.async_copy(
                    ct.at[sl], cos_out.at[b, pl.ds(half * HALF, HALF),
                                          pl.ds(s0, CHUNK)], csem.at[sl]))
                wops[sl].append(pltpu.async_copy(
                    st.at[sl], sin_out.at[b, pl.ds(half * HALF, HALF),
                                          pl.ds(s0, CHUNK)], ssem.at[sl]))
        for sl in range(NSLOT):
            for op in wops[sl]:
                op.wait()

    return k(fused_table, position_ids)


def kernel(x, position_ids, cos_cached, sin_cached):
    fused = jnp.concatenate([cos_cached, sin_cached], axis=-1)
    cos_t, sin_t = _rope_gather(position_ids, fused)
    cos = jnp.swapaxes(cos_t, 1, 2).astype(x.dtype)
    sin = jnp.swapaxes(sin_t, 1, 2).astype(x.dtype)
    return cos, sin


# speedup vs baseline: 2.2549x; 1.4296x over previous
import functools

import jax
import jax.numpy as jnp
from jax import lax
from jax.experimental import pallas as pl
from jax.experimental.pallas import tpu as pltpu
from jax.experimental.pallas import tpu_sc as plsc

DIM = 64
HALF = 32
CHUNK = 128
NSLOT = 2


def _rope_gather(position_ids, fused_table):
    bsz, seq = position_ids.shape
    n_total = bsz * seq
    info = plsc.get_sparse_core_info()
    nw = info.num_cores * info.num_subcores
    n_per_w = n_total // nw
    n_ch = n_per_w // CHUNK
    w_per_b = nw // bsz

    mesh = plsc.VectorSubcoreMesh(core_axis_name="c", subcore_axis_name="s")

    @functools.partial(
        pl.kernel,
        mesh=mesh,
        compiler_params=pltpu.CompilerParams(
            use_tc_tiling_on_sc=True, needs_layout_passes=False),
        out_type=(
            jax.ShapeDtypeStruct((bsz, DIM, seq), jnp.float32),
            jax.ShapeDtypeStruct((bsz, DIM, seq), jnp.float32),
        ),
        scratch_types=[
            pltpu.VMEM((n_per_w,), jnp.int32),
            pltpu.VMEM((NSLOT, CHUNK, 2 * DIM), jnp.float32),
            pltpu.VMEM((NSLOT, HALF, CHUNK), jnp.float32),
            pltpu.VMEM((NSLOT, HALF, CHUNK), jnp.float32),
            pltpu.SemaphoreType.DMA((NSLOT,)),
            pltpu.SemaphoreType.DMA((NSLOT,)),
            pltpu.SemaphoreType.DMA((NSLOT,)),
        ],
    )
    def k(tab_hbm, idx_hbm, cos_out, sin_out, idx_v, buf, ct, st, gsem, csem, ssem):
        wid = lax.axis_index("s") * info.num_cores + lax.axis_index("c")
        b = wid // w_per_b
        s_base = (wid % w_per_b) * n_per_w
        pltpu.sync_copy(idx_hbm.at[b, pl.ds(s_base, n_per_w)], idx_v)

        iota = lax.iota(jnp.int32, 16)
        gops = [None] * NSLOT
        wops = [[] for _ in range(NSLOT)]

        def refill(r):
            sl = r % NSLOT
            gops[sl] = pltpu.async_copy(
                tab_hbm.at[idx_v.at[pl.ds(r * CHUNK, CHUNK)]], buf.at[sl],
                gsem.at[sl])

        def transpose(sl):
            src = buf.at[sl]
            cdst = ct.at[sl]
            sdst = st.at[sl]

            @plsc.parallel_loop(0, HALF, step=1, unroll=4)
            def body(d):
                ccol = jnp.zeros((16,), jnp.int32) + d
                scol = ccol + DIM
                for g in range(8):
                    rows = iota + (g * 16)
                    cdst[d, pl.ds(g * 16, 16)] = plsc.load_gather(
                        src, [rows, ccol])
                    sdst[d, pl.ds(g * 16, 16)] = plsc.load_gather(
                        src, [rows, scol])

        refill(0)
        for j in range(n_ch):
            sl = j % NSLOT
            gops[sl].wait()
            if j + 1 < n_ch:
                refill(j + 1)
            for op in wops[sl]:
                op.wait()
            wops[sl] = []
            transpose(sl)
            s0 = s_base + j * CHUNK
            for half in range(2):
                wops[sl].append(pltpu.async_copy(
                    ct.at[sl], cos_out.at[b, pl.ds(half * HALF, HALF),
                                          pl.ds(s0, CHUNK)], csem.at[sl]))
                wops[sl].append(pltpu.async_copy(
                    st.at[sl], sin_out.at[b, pl.ds(half * HALF, HALF),
                                          pl.ds(s0, CHUNK)], ssem.at[sl]))
        for sl in range(NSLOT):
            for op in wops[sl]:
                op.wait()

    return k(fused_table, position_ids)


def kernel(x, position_ids, cos_cached, sin_cached):
    fused = jnp.concatenate([cos_cached, sin_cached], axis=-1)
    cos_t, sin_t = _rope_gather(position_ids, fused)
    cos = jnp.swapaxes(cos_t, 1, 2).astype(x.dtype)
    sin = jnp.swapaxes(sin_t, 1, 2).astype(x.dtype)
    return cos, sin


# speedup vs baseline: 4.2063x; 1.8654x over previous
import functools

import jax
import jax.numpy as jnp
from jax import lax
from jax.experimental import pallas as pl
from jax.experimental.pallas import tpu as pltpu
from jax.experimental.pallas import tpu_sc as plsc

DIM = 64
HALF = 32
DGRP = 2
CHUNK = 2048
NSLOT = 2


def _rope_gather(position_ids, cos_t, sin_t):
    bsz, seq = position_ids.shape
    info = plsc.get_sparse_core_info()
    nw = info.num_cores * info.num_subcores
    n_grp = HALF // DGRP
    b_grp = nw // n_grp
    b_per_w = bsz // b_grp
    n_ch = seq // CHUNK

    mesh = plsc.VectorSubcoreMesh(core_axis_name="c", subcore_axis_name="s")

    @functools.partial(
        pl.kernel,
        mesh=mesh,
        compiler_params=pltpu.CompilerParams(
            use_tc_tiling_on_sc=True, needs_layout_passes=False),
        out_type=(
            jax.ShapeDtypeStruct((bsz, DIM, seq), jnp.float32),
            jax.ShapeDtypeStruct((bsz, DIM, seq), jnp.float32),
        ),
        scratch_types=[
            pltpu.VMEM((2 * DGRP, seq), jnp.float32),
            pltpu.VMEM((b_per_w, seq), jnp.int32),
            pltpu.VMEM((NSLOT, 2, DGRP, CHUNK), jnp.float32),
            pltpu.SemaphoreType.DMA,
            pltpu.SemaphoreType.DMA((NSLOT,)),
        ],
    )
    def k(cos_hbm, sin_hbm, idx_hbm, cos_out, sin_out,
          rows_v, idx_v, obuf, rsem, osem):
        wid = lax.axis_index("s") * info.num_cores + lax.axis_index("c")
        g = wid // b_grp
        bq = wid % b_grp
        d0 = g * DGRP
        b0 = bq * b_per_w

        ops = [
            pltpu.async_copy(cos_hbm.at[pl.ds(d0, DGRP), :],
                             rows_v.at[pl.ds(0, DGRP), :], rsem),
            pltpu.async_copy(sin_hbm.at[pl.ds(d0, DGRP), :],
                             rows_v.at[pl.ds(DGRP, DGRP), :], rsem),
            pltpu.async_copy(idx_hbm.at[pl.ds(b0, b_per_w), :], idx_v, rsem),
        ]
        for op in ops:
            op.wait()

        wops = [[] for _ in range(NSLOT)]
        for bi in range(b_per_w):
            for j in range(n_ch):
                sl = (bi * n_ch + j) % NSLOT
                for op in wops[sl]:
                    op.wait()
                wops[sl] = []
                ob = obuf.at[sl]
                s0 = j * CHUNK

                @plsc.parallel_loop(0, CHUNK // 16, step=1, unroll=4)
                def body(i):
                    idxv = idx_v[bi, pl.ds(s0 + i * 16, 16)]
                    for tt in range(2):
                        for dd in range(DGRP):
                            r = jnp.zeros((16,), jnp.int32) + (tt * DGRP + dd)
                            ob[tt, dd, pl.ds(i * 16, 16)] = plsc.load_gather(
                                rows_v, [r, idxv])

                b = b0 + bi
                for half in range(2):
                    dh = half * HALF + d0
                    wops[sl].append(pltpu.async_copy(
                        ob.at[0], cos_out.at[b, pl.ds(dh, DGRP), pl.ds(s0, CHUNK)],
                        osem.at[sl]))
                    wops[sl].append(pltpu.async_copy(
                        ob.at[1], sin_out.at[b, pl.ds(dh, DGRP), pl.ds(s0, CHUNK)],
                        osem.at[sl]))
        for sl in range(NSLOT):
            for op in wops[sl]:
                op.wait()

    return k(cos_t, sin_t, position_ids)


def kernel(x, position_ids, cos_cached, sin_cached):
    cos_t = jnp.swapaxes(cos_cached, 0, 1)
    sin_t = jnp.swapaxes(sin_cached, 0, 1)
    cos_o, sin_o = _rope_gather(position_ids, cos_t, sin_t)
    cos = jnp.swapaxes(cos_o, 1, 2).astype(x.dtype)
    sin = jnp.swapaxes(sin_o, 1, 2).astype(x.dtype)
    return cos, sin


# speedup vs baseline: 4.2656x; 1.0141x over previous
import functools

import jax
import jax.numpy as jnp
from jax import lax
from jax.experimental import pallas as pl
from jax.experimental.pallas import tpu as pltpu
from jax.experimental.pallas import tpu_sc as plsc

DIM = 64
HALF = 32
DGRP = 2
CHUNK = 2048
NSLOT = 2


def _rope_gather(position_ids, cos_t, sin_t):
    bsz, seq = position_ids.shape
    info = plsc.get_sparse_core_info()
    nw = info.num_cores * info.num_subcores
    n_grp = HALF // DGRP
    b_grp = nw // n_grp
    b_per_w = bsz // b_grp
    n_ch = seq // CHUNK

    mesh = plsc.VectorSubcoreMesh(core_axis_name="c", subcore_axis_name="s")

    @functools.partial(
        pl.kernel,
        mesh=mesh,
        compiler_params=pltpu.CompilerParams(
            use_tc_tiling_on_sc=True, needs_layout_passes=False),
        out_type=(
            jax.ShapeDtypeStruct((bsz, DIM, seq), jnp.float32),
            jax.ShapeDtypeStruct((bsz, DIM, seq), jnp.float32),
        ),
        scratch_types=[
            pltpu.VMEM((2 * DGRP * seq,), jnp.float32),
            pltpu.VMEM((b_per_w * seq,), jnp.int32),
            pltpu.VMEM((NSLOT, 2, DGRP, CHUNK), jnp.float32),
            pltpu.SemaphoreType.DMA,
            pltpu.SemaphoreType.DMA((NSLOT,)),
        ],
    )
    def k(cos_hbm, sin_hbm, idx_hbm, cos_out, sin_out,
          rows_v, idx_v, obuf, rsem, osem):
        wid = lax.axis_index("s") * info.num_cores + lax.axis_index("c")
        g = wid // b_grp
        bq = wid % b_grp
        d0 = g * DGRP
        b0 = bq * b_per_w

        ops = []
        for dd in range(DGRP):
            ops.append(pltpu.async_copy(
                cos_hbm.at[d0 + dd, :], rows_v.at[pl.ds(dd * seq, seq)], rsem))
            ops.append(pltpu.async_copy(
                sin_hbm.at[d0 + dd, :],
                rows_v.at[pl.ds((DGRP + dd) * seq, seq)], rsem))
        for bi in range(b_per_w):
            ops.append(pltpu.async_copy(
                idx_hbm.at[b0 + bi, :], idx_v.at[pl.ds(bi * seq, seq)], rsem))
        for op in ops:
            op.wait()

        wops = [[] for _ in range(NSLOT)]
        for bi in range(b_per_w):
            for j in range(n_ch):
                sl = (bi * n_ch + j) % NSLOT
                for op in wops[sl]:
                    op.wait()
                wops[sl] = []
                ob = obuf.at[sl]
                s0 = j * CHUNK

                @plsc.parallel_loop(0, CHUNK // 16, step=1, unroll=4)
                def body(i):
                    idxv = idx_v[pl.ds(bi * seq + s0 + i * 16, 16)]
                    for tt in range(2):
                        for dd in range(DGRP):
                            flat = idxv + ((tt * DGRP + dd) * seq)
                            ob[tt, dd, pl.ds(i * 16, 16)] = plsc.load_gather(
                                rows_v, [flat])

                b = b0 + bi
                for half in range(2):
                    dh = half * HALF + d0
                    wops[sl].append(pltpu.async_copy(
                        ob.at[0], cos_out.at[b, pl.ds(dh, DGRP), pl.ds(s0, CHUNK)],
                        osem.at[sl]))
                    wops[sl].append(pltpu.async_copy(
                        ob.at[1], sin_out.at[b, pl.ds(dh, DGRP), pl.ds(s0, CHUNK)],
                        osem.at[sl]))
        for sl in range(NSLOT):
            for op in wops[sl]:
                op.wait()

    return k(cos_t, sin_t, position_ids)


def kernel(x, position_ids, cos_cached, sin_cached):
    cos_t = jnp.swapaxes(cos_cached, 0, 1)
    sin_t = jnp.swapaxes(sin_cached, 0, 1)
    cos_o, sin_o = _rope_gather(position_ids, cos_t, sin_t)
    cos = jnp.swapaxes(cos_o, 1, 2).astype(x.dtype)
    sin = jnp.swapaxes(sin_o, 1, 2).astype(x.dtype)
    return cos, sin
